# baseline (speedup 1.0000x reference)
import functools

import jax
import jax.numpy as jnp
from jax import lax
from jax.experimental import pallas as pl
from jax.experimental.pallas import tpu as pltpu
from jax.experimental.pallas import tpu_sc as plsc

NUM_FIELDS = 26
VOCAB = 100000
EMBED_DIM = 32
BATCH = 4096
DENSE_NUM = 13

_NC = 2
_NS = 16
_BW = 2048
_BSH = 11
_CSH = 14
_NFULL = VOCAB // _BW
_NCOARSE = 7
_TAILS = _NFULL * _BW
_TAILW = 1664
_LASTS = _TAILS + _TAILW
_NBKT = _NFULL + 2

_mesh = plsc.VectorSubcoreMesh(core_axis_name="c", subcore_axis_name="s")
_iota16 = lambda: lax.iota(jnp.int32, 16)


@functools.partial(
    pl.kernel,
    mesh=_mesh,
    out_type=jax.ShapeDtypeStruct(
        (NUM_FIELDS * EMBED_DIM + EMBED_DIM, BATCH), jnp.float32),
    scratch_types=[
        pltpu.VMEM((EMBED_DIM, BATCH // EMBED_DIM), jnp.int32),
        pltpu.VMEM((4224,), jnp.int32),
        pltpu.VMEM((4224,), jnp.int32),
        pltpu.VMEM((4224,), jnp.int32),
        pltpu.VMEM((4224,), jnp.int32),
        pltpu.VMEM((8, _BW), jnp.float32),
        pltpu.VMEM((8, _BW), jnp.float32),
        pltpu.VMEM((8, _BW), jnp.float32),
        pltpu.VMEM((8, _BW), jnp.float32),
        pltpu.VMEM((8, BATCH), jnp.float32),
        pltpu.VMEM((EMBED_DIM, 128), jnp.float32),
        pltpu.SMEM((_NCOARSE + 1,), jnp.int32),
        pltpu.SMEM((_NBKT + 1,), jnp.int32),
        pltpu.SemaphoreType.DMA,
        pltpu.SemaphoreType.DMA,
        pltpu.SemaphoreType.DMA,
        pltpu.SemaphoreType.DMA,
    ],
    compiler_params=pltpu.CompilerParams(
        use_tc_tiling_on_sc=True, needs_layout_passes=False),
)
def _sc_gather(idx_hbm, tbl_hbm, last_hbm, dense_hbm, out_hbm, idx_v,
               cidx_v, cpos_v, bidx_v, bpos_v, ch0, ch1, ch2, ch3, orow_v,
               last_v, coff_s, boff_s, sem0, sem1, sem2, sem3):
    wid = lax.axis_index("s") * _NC + lax.axis_index("c")
    f = wid
    bufs = ((ch0, sem0), (ch1, sem1), (ch2, sem2), (ch3, sem3))

    @pl.when(jnp.logical_and(wid >= NUM_FIELDS, wid < NUM_FIELDS + 4))
    def _dense_copy():
        j = wid - NUM_FIELDS
        pltpu.sync_copy(dense_hbm.at[pl.ds(j * 8, 8)], orow_v)
        pltpu.sync_copy(
            orow_v, out_hbm.at[pl.ds(NUM_FIELDS * EMBED_DIM + j * 8, 8)])

    @pl.when(wid < NUM_FIELDS)
    def _body():
        def _chunk_src(d8, k, width):
            return tbl_hbm.at[f, pl.ds(d8 * 8, 8), pl.ds(k * _BW, width)]

        for p, (ch, sem) in enumerate(bufs):
            pltpu.make_async_copy(_chunk_src(0, p, _BW), ch, sem).start()
        pltpu.sync_copy(idx_hbm.at[f], idx_v)
        pltpu.sync_copy(last_hbm.at[f], last_v)
        iota = _iota16()

        with jax.named_scope("coarse"):
            def _coarse(k, off_vec):
                def _row(r, off_vec):
                    for s in range(8):
                        v = idx_v[r, pl.ds(s * 16, 16)]
                        m = lax.shift_right_logical(v, _CSH) == k
                        cnt = plsc.all_reduce_population_count(m)
                        rank = plsc.cumsum(jnp.where(m, 1, 0)) - 1
                        dst = off_vec + rank
                        pos = r * 128 + s * 16 + iota
                        plsc.store_scatter(cidx_v, [dst], v, mask=m)
                        plsc.store_scatter(cpos_v, [dst], pos, mask=m)
                        off_vec = off_vec + cnt
                    return off_vec
                off_vec = lax.fori_loop(0, EMBED_DIM, _row, off_vec)
                coff_s[k + 1] = jnp.max(off_vec)
                return off_vec

            coff_s[0] = 0
            boff_s[0] = 0
            lax.fori_loop(0, _NCOARSE, _coarse, jnp.zeros((16,), jnp.int32))

        def _refine(c, nsub, fine_of):
            cstart = coff_s[c]
            cend = coff_s[c + 1]
            nvec = lax.div(cend - cstart + 15, 16)

            def _sub(w):
                kf = c * 8 + w

                def _vec(v, off_vec):
                    o = cstart + v * 16
                    inr = (o + iota) < cend
                    vv = cidx_v[pl.ds(o, 16)]
                    pp = cpos_v[pl.ds(o, 16)]
                    m = jnp.logical_and(inr, fine_of(vv) == w)
                    cnt = plsc.all_reduce_population_count(m)
                    rank = plsc.cumsum(jnp.where(m, 1, 0)) - 1
                    dst = off_vec + rank
                    plsc.store_scatter(bidx_v, [dst], vv, mask=m)
                    plsc.store_scatter(bpos_v, [dst], pp, mask=m)
                    return off_vec + cnt

                off0 = jnp.full((16,), boff_s[kf], jnp.int32)
                offe = lax.fori_loop(0, nvec, _vec, off0)
                boff_s[kf + 1] = jnp.max(offe)

            for w in range(nsub):
                _sub(w)

        def _process(k, ch, base, dlo):
            start = boff_s[k]
            end = boff_s[k + 1]

            def _vec(v, carry):
                o = start + v * 16
                m = (o + iota) < end
                lidx = bidx_v[pl.ds(o, 16)] - base
                pos = bpos_v[pl.ds(o, 16)]
                for d in range(8):
                    dsp = jnp.full((16,), dlo + d, jnp.int32)
                    osp = jnp.full((16,), d, jnp.int32)
                    val = plsc.load_gather(ch, [dsp, lidx], mask=m)
                    plsc.store_scatter(orow_v, [osp, pos], val, mask=m)
                return carry

            nvec = lax.div(end - start + 15, 16)
            lax.fori_loop(0, nvec, _vec, 0)

        def _fine_id(vv):
            return jnp.bitwise_and(lax.shift_right_logical(vv, _BSH), 7)

        def _tail_id(vv):
            return jnp.where(vv >= _LASTS, 1, 0)

        for d8 in range(4):
          with jax.named_scope(f"scan_d8_{d8}"):
            if d8 > 0:
                for p, (ch, sem) in enumerate(bufs):
                    pltpu.make_async_copy(_chunk_src(d8, p, _BW), ch, sem).start()
            else:
                _refine(0, 8, _fine_id)

            def _coarse_step(c, carry):
                if d8 == 0:
                    @pl.when(c < _NCOARSE - 2)
                    def _():
                        _refine(c + 1, 8, _fine_id)
                for w in range(8):
                    k = c * 8 + w
                    ch, sem = bufs[w % 4]
                    pltpu.make_async_copy(_chunk_src(d8, k, _BW), ch, sem).wait()
                    _process(k, ch, k * _BW, 0)

                    @pl.when(k + 4 < _NFULL)
                    def _():
                        pltpu.make_async_copy(
                            _chunk_src(d8, k + 4, _BW), ch, sem).start()
                return carry

            lax.fori_loop(0, _NCOARSE - 1, _coarse_step, 0)
            if d8 == 0:
                _refine(6, 2, _tail_id)
            pltpu.sync_copy(_chunk_src(d8, _NFULL, _TAILW),
                            ch0.at[:, pl.ds(0, _TAILW)])
            _process(_NFULL, ch0, _TAILS, 0)
            _process(_NFULL + 1, last_v, VOCAB - 128, d8 * 8)
            pltpu.sync_copy(orow_v, out_hbm.at[pl.ds(f * EMBED_DIM + d8 * 8, 8)])


def _dense_body(x_ref, w_ref, b_ref, o_ref):
    o_ref[...] = lax.dot_general(
        w_ref[...], x_ref[...], (((1,), (1,)), ((), ())),
        preferred_element_type=jnp.float32,
    ) + b_ref[...]


def _dense_tc(x, w, b2d):
    return pl.pallas_call(
        _dense_body,
        out_shape=jax.ShapeDtypeStruct((EMBED_DIM, BATCH), jnp.float32),
    )(x, w, b2d)


def kernel(sparse_indices, dense_features, tables, W, b):
    tbl_t = jnp.transpose(tables, (0, 2, 1))
    last_t = jnp.transpose(tables[:, VOCAB - 128:, :], (0, 2, 1))
    idx3d = jnp.transpose(sparse_indices, (1, 0)).reshape(
        NUM_FIELDS, EMBED_DIM, BATCH // EMBED_DIM)
    dense_t = _dense_tc(dense_features, W, b[:, None])
    out_t = _sc_gather(idx3d, tbl_t, last_t, dense_t)
    return jnp.transpose(out_t, (1, 0))


# speedup vs baseline: 1.0553x; 1.0553x over previous
import functools

import jax
import jax.numpy as jnp
from jax import lax
from jax.experimental import pallas as pl
from jax.experimental.pallas import tpu as pltpu
from jax.experimental.pallas import tpu_sc as plsc

NUM_FIELDS = 26
VOCAB = 100000
EMBED_DIM = 32
BATCH = 4096
DENSE_NUM = 13

_NC = 2
_NS = 16
_BW = 2048
_BSH = 11
_CSH = 14
_NFULL = VOCAB // _BW
_NCOARSE = 7
_TAILS = _NFULL * _BW
_TAILW = 1664
_LASTS = _TAILS + _TAILW
_NBKT = _NFULL + 2

_mesh = plsc.VectorSubcoreMesh(core_axis_name="c", subcore_axis_name="s")
_iota16 = lambda: lax.iota(jnp.int32, 16)


@functools.partial(
    pl.kernel,
    mesh=_mesh,
    out_type=jax.ShapeDtypeStruct(
        (NUM_FIELDS * EMBED_DIM + EMBED_DIM, BATCH), jnp.float32),
    scratch_types=[
        pltpu.VMEM((EMBED_DIM, BATCH // EMBED_DIM), jnp.int32),
        pltpu.VMEM((4224,), jnp.int32),
        pltpu.VMEM((4224,), jnp.int32),
        pltpu.VMEM((4224,), jnp.int32),
        pltpu.VMEM((4224,), jnp.int32),
        pltpu.VMEM((8, _BW), jnp.float32),
        pltpu.VMEM((8, _BW), jnp.float32),
        pltpu.VMEM((8, _BW), jnp.float32),
        pltpu.VMEM((8, _BW), jnp.float32),
        pltpu.VMEM((8, BATCH), jnp.float32),
        pltpu.VMEM((EMBED_DIM, 128), jnp.float32),
        pltpu.SMEM((_NCOARSE + 1,), jnp.int32),
        pltpu.SMEM((_NBKT + 1,), jnp.int32),
        pltpu.SemaphoreType.DMA,
        pltpu.SemaphoreType.DMA,
        pltpu.SemaphoreType.DMA,
        pltpu.SemaphoreType.DMA,
    ],
    compiler_params=pltpu.CompilerParams(
        use_tc_tiling_on_sc=True, needs_layout_passes=False),
)
def _sc_gather(idx_hbm, tbl_hbm, last_hbm, dense_hbm, out_hbm, idx_v,
               cidx_v, cpos_v, bidx_v, bpos_v, ch0, ch1, ch2, ch3, orow_v,
               last_v, coff_s, boff_s, sem0, sem1, sem2, sem3):
    wid = lax.axis_index("s") * _NC + lax.axis_index("c")
    f = wid
    bufs = ((ch0, sem0), (ch1, sem1), (ch2, sem2), (ch3, sem3))

    @pl.when(jnp.logical_and(wid >= NUM_FIELDS, wid < NUM_FIELDS + 4))
    def _dense_copy():
        j = wid - NUM_FIELDS
        pltpu.sync_copy(dense_hbm.at[pl.ds(j * 8, 8)], orow_v)
        pltpu.sync_copy(
            orow_v, out_hbm.at[pl.ds(NUM_FIELDS * EMBED_DIM + j * 8, 8)])

    @pl.when(wid < NUM_FIELDS)
    def _body():
        def _chunk_src(d8, k, width):
            return tbl_hbm.at[f, pl.ds(d8 * 8, 8), pl.ds(k * _BW, width)]

        for p, (ch, sem) in enumerate(bufs):
            pltpu.make_async_copy(_chunk_src(0, p, _BW), ch, sem).start()
        pltpu.sync_copy(idx_hbm.at[f], idx_v)
        pltpu.sync_copy(last_hbm.at[f], last_v)
        iota = _iota16()

        with jax.named_scope("coarse"):
            def _coarse(k, off_vec):
                def _row(r, off_vec):
                    for s in range(8):
                        v = idx_v[r, pl.ds(s * 16, 16)]
                        m = lax.shift_right_logical(v, _CSH) == k
                        cnt = plsc.all_reduce_population_count(m)
                        rank = plsc.cumsum(jnp.where(m, 1, 0)) - 1
                        dst = off_vec + rank
                        pos = r * 128 + s * 16 + iota
                        plsc.store_scatter(cidx_v, [dst], v, mask=m)
                        plsc.store_scatter(cpos_v, [dst], pos, mask=m)
                        off_vec = off_vec + cnt
                    return off_vec
                off_vec = lax.fori_loop(0, EMBED_DIM, _row, off_vec)
                coff_s[k + 1] = jnp.max(off_vec)
                return off_vec

            coff_s[0] = 0
            boff_s[0] = 0
            lax.fori_loop(0, _NCOARSE, _coarse, jnp.zeros((16,), jnp.int32))

        def _refine_sub(c, w, fine_of):
            cstart = coff_s[c]
            cend = coff_s[c + 1]
            nvec = lax.div(cend - cstart + 15, 16)
            kf = c * 8 + w

            def _vec(v, off_vec):
                o = cstart + v * 16
                inr = (o + iota) < cend
                vv = cidx_v[pl.ds(o, 16)]
                pp = cpos_v[pl.ds(o, 16)]
                m = jnp.logical_and(inr, fine_of(vv) == w)
                cnt = plsc.all_reduce_population_count(m)
                rank = plsc.cumsum(jnp.where(m, 1, 0)) - 1
                dst = off_vec + rank
                plsc.store_scatter(bidx_v, [dst], vv, mask=m)
                plsc.store_scatter(bpos_v, [dst], pp, mask=m)
                return off_vec + cnt

            off0 = jnp.full((16,), boff_s[kf], jnp.int32)
            offe = lax.fori_loop(0, nvec, _vec, off0)
            boff_s[kf + 1] = jnp.max(offe)

        def _refine(c, nsub, fine_of):
            for w in range(nsub):
                _refine_sub(c, w, fine_of)

        def _process(k, ch, base, dlo):
            start = boff_s[k]
            end = boff_s[k + 1]

            def _vec(v, carry):
                o = start + v * 16
                m = (o + iota) < end
                lidx = bidx_v[pl.ds(o, 16)] - base
                pos = bpos_v[pl.ds(o, 16)]
                for d in range(8):
                    dsp = jnp.full((16,), dlo + d, jnp.int32)
                    osp = jnp.full((16,), d, jnp.int32)
                    val = plsc.load_gather(ch, [dsp, lidx], mask=m)
                    plsc.store_scatter(orow_v, [osp, pos], val, mask=m)
                return carry

            nvec = lax.div(end - start + 15, 16)
            lax.fori_loop(0, nvec, _vec, 0)

        def _fine_id(vv):
            return jnp.bitwise_and(lax.shift_right_logical(vv, _BSH), 7)

        def _tail_id(vv):
            return jnp.where(vv >= _LASTS, 1, 0)

        for d8 in range(4):
          with jax.named_scope(f"scan_d8_{d8}"):
            if d8 > 0:
                for p, (ch, sem) in enumerate(bufs):
                    pltpu.make_async_copy(_chunk_src(d8, p, _BW), ch, sem).start()
            else:
                _refine(0, 8, _fine_id)

            def _coarse_step(c, carry):
                for w in range(8):
                    k = c * 8 + w
                    ch, sem = bufs[w % 4]
                    pltpu.make_async_copy(_chunk_src(d8, k, _BW), ch, sem).wait()
                    _process(k, ch, k * _BW, 0)

                    @pl.when(k + 4 < _NFULL)
                    def _():
                        pltpu.make_async_copy(
                            _chunk_src(d8, k + 4, _BW), ch, sem).start()
                    if d8 == 0:
                        @pl.when(c < _NCOARSE - 2)
                        def _():
                            _refine_sub(c + 1, w, _fine_id)
                return carry

            lax.fori_loop(0, _NCOARSE - 1, _coarse_step, 0)
            if d8 == 0:
                _refine(6, 2, _tail_id)
            pltpu.sync_copy(_chunk_src(d8, _NFULL, _TAILW),
                            ch0.at[:, pl.ds(0, _TAILW)])
            _process(_NFULL, ch0, _TAILS, 0)
            _process(_NFULL + 1, last_v, VOCAB - 128, d8 * 8)
            pltpu.sync_copy(orow_v, out_hbm.at[pl.ds(f * EMBED_DIM + d8 * 8, 8)])


def _dense_body(x_ref, w_ref, b_ref, o_ref):
    o_ref[...] = lax.dot_general(
        w_ref[...], x_ref[...], (((1,), (1,)), ((), ())),
        preferred_element_type=jnp.float32,
    ) + b_ref[...]


def _dense_tc(x, w, b2d):
    return pl.pallas_call(
        _dense_body,
        out_shape=jax.ShapeDtypeStruct((EMBED_DIM, BATCH), jnp.float32),
    )(x, w, b2d)


def kernel(sparse_indices, dense_features, tables, W, b):
    tbl_t = jnp.transpose(tables, (0, 2, 1))
    last_t = jnp.transpose(tables[:, VOCAB - 128:, :], (0, 2, 1))
    idx3d = jnp.transpose(sparse_indices, (1, 0)).reshape(
        NUM_FIELDS, EMBED_DIM, BATCH // EMBED_DIM)
    dense_t = _dense_tc(dense_features, W, b[:, None])
    out_t = _sc_gather(idx3d, tbl_t, last_t, dense_t)
    return jnp.transpose(out_t, (1, 0))
